# baseline (speedup 1.0000x reference)
import functools

import jax
import jax.numpy as jnp
from jax import lax
from jax.experimental import pallas as pl
from jax.experimental.pallas import tpu as pltpu
from jax.experimental.pallas import tpu_sc as plsc

N_PERTS = 1000000
N_LATENT = 64
B = 16384
M = 2
PADDING_IDX = 0

NC = 2
NS = 16
NW = NC * NS
PER_W = B // NW
K = PER_W * M
CH = 128
NCH = K // CH
L = 16


def _sc_kernel(perts_hbm, dos_hbm, emb_hbm, beta_hbm, bias_hbm, out_hbm,
               idx_v, dos_v, betag_v, biasg_v, rows_v, coeff_v, out_v, sem):
    wid = lax.axis_index("s") * NC + lax.axis_index("c")
    row0 = wid * NCH

    pltpu.sync_copy(perts_hbm.at[pl.ds(row0, NCH)], idx_v)
    pltpu.sync_copy(dos_hbm.at[pl.ds(row0, NCH)], dos_v)

    copies = []
    for j in range(NCH):
        idx_j = idx_v.at[j]
        copies.append(pltpu.async_copy(
            emb_hbm.at[idx_j], rows_v.at[pl.ds(j * CH, CH)], sem))
        copies.append(pltpu.async_copy(beta_hbm.at[idx_j], betag_v.at[j], sem))
        copies.append(pltpu.async_copy(bias_hbm.at[idx_j], biasg_v.at[j], sem))
    for c in copies:
        c.wait()

    for j in range(NCH):
        for oi in range(CH // L):
            o = oi * L
            d = dos_v[j, pl.ds(o, L)]
            bg = betag_v[j, pl.ds(o, L)]
            hg = biasg_v[j, pl.ds(o, L)]
            p = idx_v[j, pl.ds(o, L)]
            t = d / (d + 2.0)
            t2 = t * t
            l1p = 2.0 * t * (1.0 + t2 * (1.0 / 3.0 + t2 * (
                0.2 + t2 * (1.0 / 7.0 + t2 * (1.0 / 9.0)))))
            z = l1p * bg + hg
            s = 1.0 / (1.0 + jnp.exp(-z))
            s0 = 1.0 / (1.0 + jnp.exp(-hg))
            c = jnp.where(p == PADDING_IDX, 0.0, s - s0)
            coeff_v[pl.ds((j * (CH // L) + oi) * L, L)] = c

    def body(i, _):
        k0 = 2 * i
        cv = coeff_v[pl.ds(k0, L)]
        c0 = cv[0]
        c1 = cv[1]
        r0 = rows_v.at[k0]
        r1 = rows_v.at[k0 + 1]
        o = out_v.at[i]
        for q in range(N_LATENT // L):
            sl = pl.ds(q * L, L)
            o[sl] = c0 * r0[sl] + c1 * r1[sl]
        return 0

    lax.fori_loop(0, PER_W, body, 0)

    pltpu.sync_copy(out_v, out_hbm.at[pl.ds(wid * PER_W, PER_W)])


@jax.jit
def kernel(perts, dosages, embedding, beta, bias):
    perts2d = perts.astype(jnp.int32).reshape(B * M // CH, CH)
    dos2d = dosages.astype(jnp.float32).reshape(B * M // CH, CH)
    beta_f = beta.reshape(N_PERTS)
    bias_f = bias.reshape(N_PERTS)

    mesh = plsc.VectorSubcoreMesh(core_axis_name="c", subcore_axis_name="s")
    fn = functools.partial(
        pl.kernel,
        mesh=mesh,
        compiler_params=pltpu.CompilerParams(use_tc_tiling_on_sc=False),
        out_type=jax.ShapeDtypeStruct((B, N_LATENT), jnp.float32),
        scratch_types=[
            pltpu.VMEM((NCH, CH), jnp.int32),
            pltpu.VMEM((NCH, CH), jnp.float32),
            pltpu.VMEM((NCH, CH), jnp.float32),
            pltpu.VMEM((NCH, CH), jnp.float32),
            pltpu.VMEM((K, N_LATENT), jnp.float32),
            pltpu.VMEM((K,), jnp.float32),
            pltpu.VMEM((PER_W, N_LATENT), jnp.float32),
            pltpu.SemaphoreType.DMA,
        ],
    )(_sc_kernel)
    return fn(perts2d, dos2d, embedding, beta_f, bias_f)
